# Initial kernel scaffold; baseline (speedup 1.0000x reference)
import functools

import jax
import jax.numpy as jnp
from jax import lax
from jax.experimental import pallas as pl
from jax.experimental.pallas import tpu as pltpu
from jax.experimental.pallas import tpu_sc as plsc

N_Q = 10000
N_L = 2048
E = 320000
C = 128
AGGW = 144

NC = 2
NS = 16
NW = NC * NS
EPW = E // NW
CE = 80
NCHUNK = EPW // CE
QPT = N_Q // NS
ZB = 125

_MESH = plsc.VectorSubcoreMesh(core_axis_name="c", subcore_axis_name="s")


@functools.partial(
    pl.kernel,
    out_type=jax.ShapeDtypeStruct((E * 4,), jnp.float32),
    mesh=_MESH,
    scratch_types=[
        pltpu.VMEM((N_L, 2), jnp.float32),
        pltpu.VMEM((N_Q, 2), jnp.float32),
        pltpu.VMEM((CE,), jnp.int32),
        pltpu.VMEM((CE,), jnp.int32),
        pltpu.VMEM((CE * 4,), jnp.float32),
    ],
)
def _gather_kin(dst_hbm, src_hbm, lat_hbm, qp_hbm, kin_hbm,
                lat_v, qp_v, src_v, dst_v, kin_v):
    wid = lax.axis_index("s") * NC + lax.axis_index("c")
    base = wid * EPW
    pltpu.sync_copy(lat_hbm, lat_v)
    pltpu.sync_copy(qp_hbm, qp_v)
    zeros16 = jnp.zeros((16,), jnp.int32)
    ones16 = jnp.ones((16,), jnp.int32)
    iota4 = lax.iota(jnp.int32, 16) * 4

    def chunk_body(ci, _):
        e0 = base + ci * CE
        pltpu.sync_copy(src_hbm.at[pl.ds(e0, CE)], src_v)
        pltpu.sync_copy(dst_hbm.at[pl.ds(e0, CE)], dst_v)

        def sub(i, _):
            sv = src_v[pl.ds(i * 16, 16)]
            dv = dst_v[pl.ds(i * 16, 16)]
            y0 = plsc.load_gather(lat_v, [sv, zeros16])
            y1 = plsc.load_gather(lat_v, [sv, ones16])
            x0 = plsc.load_gather(qp_v, [dv, zeros16])
            x1 = plsc.load_gather(qp_v, [dv, ones16])
            fb = i * 64 + iota4
            plsc.store_scatter(kin_v, [fb], y0)
            plsc.store_scatter(kin_v, [fb + 1], y1)
            plsc.store_scatter(kin_v, [fb + 2], x0)
            plsc.store_scatter(kin_v, [fb + 3], x1)
            return 0

        lax.fori_loop(0, CE // 16, sub, 0)
        pltpu.sync_copy(kin_v, kin_hbm.at[pl.ds(e0 * 4, CE * 4)])
        return 0

    lax.fori_loop(0, NCHUNK, chunk_body, 0)


EB = 4000


def _edge_mlp_body(kin_ref, w1_ref, b1_ref, w2_ref, b2_ref, w3_ref, b3_ref,
                   k_ref):
    kin = kin_ref[...]
    h = jax.nn.gelu(
        jnp.dot(kin, w1_ref[...], preferred_element_type=jnp.float32)
        + b1_ref[...][None, :])
    h = jax.nn.gelu(
        jnp.dot(h, w2_ref[...], preferred_element_type=jnp.float32)
        + b2_ref[...][None, :])
    k_ref[...] = (
        jnp.dot(h, w3_ref[...], preferred_element_type=jnp.float32)
        + b3_ref[...][None, :])


def _edge_mlp(kin, w1, b1, w2, b2, w3, b3):
    full = lambda shape: pl.BlockSpec(shape, lambda i: tuple(0 for _ in shape))
    return pl.pallas_call(
        _edge_mlp_body,
        grid=(E // EB,),
        in_specs=[
            pl.BlockSpec((EB, 4), lambda i: (i, 0)),
            full((4, 64)), full((64,)),
            full((64, 64)), full((64,)),
            full((64, C)), full((C,)),
        ],
        out_specs=pl.BlockSpec((EB, C), lambda i: (i, 0)),
        out_shape=jax.ShapeDtypeStruct((E, C), jnp.float32),
    )(kin, w1, b1, w2, b2, w3, b3)


@functools.partial(
    pl.kernel,
    out_type=jax.ShapeDtypeStruct((NC * N_Q, AGGW), jnp.float32),
    mesh=_MESH,
    scratch_types=[
        pltpu.VMEM((CE,), jnp.int32),
        pltpu.VMEM((CE,), jnp.int32),
        pltpu.VMEM((CE, C), jnp.float32),
        pltpu.VMEM((CE, C), jnp.float32),
        pltpu.VMEM((CE, AGGW), jnp.float32),
        pltpu.VMEM((ZB, AGGW), jnp.float32),
        pltpu.VMEM_SHARED((N_Q, AGGW), jnp.float32),
        pltpu.SemaphoreType.DMA,
    ],
)
def _scatter_agg(dst_hbm, src_hbm, k_hbm, rnd_hbm, out_hbm,
                 dsti_v, srci_v, k_v, f_v, msg_v, z_v, agg_sh, sem):
    cid = lax.axis_index("c")
    sid = lax.axis_index("s")
    wid = sid * NC + cid
    z16 = jnp.zeros((16,), jnp.float32)
    cnt16 = (lax.iota(jnp.int32, 16) == 0).astype(jnp.float32)

    def zrow(i, _):
        for j in range(AGGW // 16):
            z_v[i, pl.ds(j * 16, 16)] = z16
        return 0

    lax.fori_loop(0, ZB, zrow, 0)

    def zcp(i, _):
        pltpu.sync_copy(z_v, agg_sh.at[pl.ds(sid * QPT + i * ZB, ZB)])
        return 0

    lax.fori_loop(0, QPT // ZB, zcp, 0)
    plsc.subcore_barrier()

    base = wid * EPW

    def chunk(ci, _):
        e0 = base + ci * CE
        pltpu.sync_copy(dst_hbm.at[pl.ds(e0, CE)], dsti_v)
        pltpu.sync_copy(src_hbm.at[pl.ds(e0, CE)], srci_v)
        pltpu.sync_copy(k_hbm.at[pl.ds(e0, CE)], k_v)
        pltpu.async_copy(rnd_hbm.at[srci_v], f_v, sem).wait()

        def erow(e, _):
            for j in range(C // 16):
                msg_v[e, pl.ds(j * 16, 16)] = (
                    k_v[e, pl.ds(j * 16, 16)] * f_v[e, pl.ds(j * 16, 16)])
            msg_v[e, pl.ds(C, 16)] = cnt16
            return 0

        lax.fori_loop(0, CE, erow, 0)
        pltpu.sync_copy(msg_v, agg_sh.at[dsti_v], add=True)
        return 0

    lax.fori_loop(0, NCHUNK, chunk, 0)
    plsc.subcore_barrier()

    def rd(i, _):
        r0 = sid * QPT + i * ZB
        pltpu.sync_copy(agg_sh.at[pl.ds(r0, ZB)],
                        out_hbm.at[pl.ds(cid * N_Q + r0, ZB)])
        return 0

    lax.fori_loop(0, QPT // ZB, rd, 0)


RB = 400


def _proj_body(a0_ref, a1_ref, w1_ref, b1_ref, w2_ref, b2_ref, out_ref):
    s = a0_ref[...] + a1_ref[...]
    agg = s[:, :C]
    cnt = jnp.sum(s[:, C:], axis=1, keepdims=True)
    decoded = agg / jnp.maximum(cnt, 1.0)
    p = jax.nn.gelu(
        jnp.dot(decoded, w1_ref[...], preferred_element_type=jnp.float32)
        + b1_ref[...][None, :])
    out_ref[...] = (
        jnp.dot(p, w2_ref[...], preferred_element_type=jnp.float32)
        + b2_ref[...][None, :])


def _proj(a0, a1, w1, b1, w2, b2):
    full = lambda shape: pl.BlockSpec(shape, lambda i: tuple(0 for _ in shape))
    return pl.pallas_call(
        _proj_body,
        grid=(N_Q // RB,),
        in_specs=[
            pl.BlockSpec((RB, AGGW), lambda i: (i, 0)),
            pl.BlockSpec((RB, AGGW), lambda i: (i, 0)),
            full((C, 256)), full((256,)),
            full((256, C)), full((C,)),
        ],
        out_specs=pl.BlockSpec((RB, C), lambda i: (i, 0)),
        out_shape=jax.ShapeDtypeStruct((N_Q, C), jnp.float32),
    )(a0, a1, w1, b1, w2, b2)


def kernel(rndata_flat, phys_pos_query, batch_idx_phys_query,
           latent_tokens_pos, latent_tokens_batch_idx, edge_index,
           Wk1, bk1, Wk2, bk2, Wk3, bk3, Wp1, bp1, Wp2, bp2):
    dst = edge_index[0]
    src = edge_index[1]
    kin = _gather_kin(dst, src, latent_tokens_pos, phys_pos_query)
    kin = kin.reshape(E, 4)
    k = _edge_mlp(kin, Wk1, bk1, Wk2, bk2, Wk3, bk3)
    agg2 = _scatter_agg(dst, src, k, rndata_flat)
    out = _proj(agg2[:N_Q], agg2[N_Q:], Wp1, bp1, Wp2, bp2)
    return out


# speedup vs baseline: 2.7246x; 2.7246x over previous
import functools

import jax
import jax.numpy as jnp
from jax import lax
from jax.experimental import pallas as pl
from jax.experimental.pallas import tpu as pltpu
from jax.experimental.pallas import tpu_sc as plsc

N_Q = 10000
N_L = 2048
E = 320000
C = 128
AGGW = 144

NC = 2
NS = 16
NW = NC * NS
EPW = E // NW
CE = 80
NCHUNK = EPW // CE
N_QP = 10240
QPT = N_QP // NS
ZB = 128

_MESH = plsc.VectorSubcoreMesh(core_axis_name="c", subcore_axis_name="s")


@functools.partial(
    pl.kernel,
    out_type=jax.ShapeDtypeStruct((E * 4,), jnp.float32),
    mesh=_MESH,
    compiler_params=pltpu.CompilerParams(needs_layout_passes=False, use_tc_tiling_on_sc=False),
    scratch_types=[
        pltpu.VMEM((N_L * 2,), jnp.float32),
        pltpu.VMEM((N_Q * 2,), jnp.float32),
        pltpu.VMEM((CE,), jnp.int32),
        pltpu.VMEM((CE,), jnp.int32),
        pltpu.VMEM((CE * 4,), jnp.float32),
    ],
)
def _gather_kin(dst_hbm, src_hbm, lat_hbm, qp_hbm, kin_hbm,
                lat_v, qp_v, src_v, dst_v, kin_v):
    wid = lax.axis_index("s") * NC + lax.axis_index("c")
    base = wid * EPW
    pltpu.sync_copy(lat_hbm, lat_v)
    pltpu.sync_copy(qp_hbm, qp_v)
    iota4 = lax.iota(jnp.int32, 16) * 4

    def chunk_body(ci, _):
        e0 = base + ci * CE
        pltpu.sync_copy(src_hbm.at[pl.ds(e0, CE)], src_v)
        pltpu.sync_copy(dst_hbm.at[pl.ds(e0, CE)], dst_v)

        def sub(i, _):
            sv = src_v[pl.ds(i * 16, 16)] * 2
            dv = dst_v[pl.ds(i * 16, 16)] * 2
            y0 = plsc.load_gather(lat_v, [sv])
            y1 = plsc.load_gather(lat_v, [sv + 1])
            x0 = plsc.load_gather(qp_v, [dv])
            x1 = plsc.load_gather(qp_v, [dv + 1])
            fb = i * 64 + iota4
            plsc.store_scatter(kin_v, [fb], y0)
            plsc.store_scatter(kin_v, [fb + 1], y1)
            plsc.store_scatter(kin_v, [fb + 2], x0)
            plsc.store_scatter(kin_v, [fb + 3], x1)
            return 0

        lax.fori_loop(0, CE // 16, sub, 0)
        pltpu.sync_copy(kin_v, kin_hbm.at[pl.ds(e0 * 4, CE * 4)])
        return 0

    lax.fori_loop(0, NCHUNK, chunk_body, 0)


EB = 4000


def _edge_mlp_body(kin_ref, w1_ref, b1_ref, w2_ref, b2_ref, w3_ref, b3_ref,
                   k_ref):
    kin = kin_ref[...]
    h = jax.nn.gelu(
        jnp.dot(kin, w1_ref[...], preferred_element_type=jnp.float32)
        + b1_ref[...][None, :])
    h = jax.nn.gelu(
        jnp.dot(h, w2_ref[...], preferred_element_type=jnp.float32)
        + b2_ref[...][None, :])
    k_ref[...] = (
        jnp.dot(h, w3_ref[...], preferred_element_type=jnp.float32)
        + b3_ref[...][None, :])


def _edge_mlp(kin, w1, b1, w2, b2, w3, b3):
    full = lambda shape: pl.BlockSpec(shape, lambda i: tuple(0 for _ in shape))
    return pl.pallas_call(
        _edge_mlp_body,
        grid=(E // EB,),
        in_specs=[
            pl.BlockSpec((EB, 4), lambda i: (i, 0)),
            full((4, 64)), full((64,)),
            full((64, 64)), full((64,)),
            full((64, C)), full((C,)),
        ],
        out_specs=pl.BlockSpec((EB, C), lambda i: (i, 0)),
        out_shape=jax.ShapeDtypeStruct((E, C), jnp.float32),
    )(kin, w1, b1, w2, b2, w3, b3)


@functools.partial(
    pl.kernel,
    out_type=jax.ShapeDtypeStruct((NC * N_QP, AGGW), jnp.float32),
    mesh=_MESH,
    compiler_params=pltpu.CompilerParams(needs_layout_passes=False, use_tc_tiling_on_sc=False),
    scratch_types=[
        pltpu.VMEM((CE,), jnp.int32),
        pltpu.VMEM((CE,), jnp.int32),
        pltpu.VMEM((CE, C), jnp.float32),
        pltpu.VMEM((CE, C), jnp.float32),
        pltpu.VMEM((CE, AGGW), jnp.float32),
        pltpu.VMEM_SHARED((N_QP, AGGW), jnp.float32),
        pltpu.SemaphoreType.DMA,
    ],
)
def _scatter_agg(dst_hbm, src_hbm, k_hbm, rnd_hbm, out_hbm,
                 dsti_v, srci_v, k_v, f_v, msg_v, agg_sh, sem):
    cid = lax.axis_index("c")
    sid = lax.axis_index("s")
    wid = sid * NC + cid
    z16 = jnp.zeros((16,), jnp.float32)
    cnt16 = (lax.iota(jnp.int32, 16) == 0).astype(jnp.float32)

    def zrow(i, _):
        for j in range(AGGW // 16):
            msg_v[i, pl.ds(j * 16, 16)] = z16
        return 0

    lax.fori_loop(0, CE, zrow, 0)

    def zcp(i, _):
        pltpu.sync_copy(msg_v, agg_sh.at[pl.ds(sid * QPT + i * CE, CE)])
        return 0

    lax.fori_loop(0, QPT // CE, zcp, 0)
    plsc.subcore_barrier()

    base = wid * EPW

    def chunk(ci, _):
        e0 = base + ci * CE
        pltpu.sync_copy(dst_hbm.at[pl.ds(e0, CE)], dsti_v)
        pltpu.sync_copy(src_hbm.at[pl.ds(e0, CE)], srci_v)
        pltpu.sync_copy(k_hbm.at[pl.ds(e0, CE)], k_v)
        pltpu.async_copy(rnd_hbm.at[srci_v], f_v, sem).wait()

        def erow(e, _):
            for j in range(C // 16):
                msg_v[e, pl.ds(j * 16, 16)] = (
                    k_v[e, pl.ds(j * 16, 16)] * f_v[e, pl.ds(j * 16, 16)])
            msg_v[e, pl.ds(C, 16)] = cnt16
            return 0

        lax.fori_loop(0, CE, erow, 0)
        pltpu.sync_copy(msg_v, agg_sh.at[dsti_v], add=True)
        return 0

    lax.fori_loop(0, NCHUNK, chunk, 0)
    plsc.subcore_barrier()

    def rd(i, _):
        r0 = sid * QPT + i * ZB
        pltpu.sync_copy(agg_sh.at[pl.ds(r0, ZB)],
                        out_hbm.at[pl.ds(cid * N_QP + r0, ZB)])
        return 0

    lax.fori_loop(0, QPT // ZB, rd, 0)


RB = 400


def _proj_body(a0_ref, a1_ref, w1_ref, b1_ref, w2_ref, b2_ref, out_ref):
    s = a0_ref[...] + a1_ref[...]
    agg = s[:, :C]
    cnt = jnp.sum(s[:, C:], axis=1, keepdims=True)
    decoded = agg / jnp.maximum(cnt, 1.0)
    p = jax.nn.gelu(
        jnp.dot(decoded, w1_ref[...], preferred_element_type=jnp.float32)
        + b1_ref[...][None, :])
    out_ref[...] = (
        jnp.dot(p, w2_ref[...], preferred_element_type=jnp.float32)
        + b2_ref[...][None, :])


def _proj(a0, a1, w1, b1, w2, b2):
    full = lambda shape: pl.BlockSpec(shape, lambda i: tuple(0 for _ in shape))
    return pl.pallas_call(
        _proj_body,
        grid=(N_Q // RB,),
        in_specs=[
            pl.BlockSpec((RB, AGGW), lambda i: (i, 0)),
            pl.BlockSpec((RB, AGGW), lambda i: (i, 0)),
            full((C, 256)), full((256,)),
            full((256, C)), full((C,)),
        ],
        out_specs=pl.BlockSpec((RB, C), lambda i: (i, 0)),
        out_shape=jax.ShapeDtypeStruct((N_Q, C), jnp.float32),
    )(a0, a1, w1, b1, w2, b2)


def kernel(rndata_flat, phys_pos_query, batch_idx_phys_query,
           latent_tokens_pos, latent_tokens_batch_idx, edge_index,
           Wk1, bk1, Wk2, bk2, Wk3, bk3, Wp1, bp1, Wp2, bp2):
    dst = edge_index[0]
    src = edge_index[1]
    kin = _gather_kin(dst, src, latent_tokens_pos.reshape(-1),
                      phys_pos_query.reshape(-1))
    kin = kin.reshape(E, 4)
    k = _edge_mlp(kin, Wk1, bk1, Wk2, bk2, Wk3, bk3)
    agg2 = _scatter_agg(dst, src, k, rndata_flat)
    out = _proj(agg2[:N_Q], agg2[N_QP:N_QP + N_Q], Wp1, bp1, Wp2, bp2)
    return out


# speedup vs baseline: 4.1507x; 1.5234x over previous
import functools

import jax
import jax.numpy as jnp
import numpy as np
from jax import lax
from jax.experimental import pallas as pl
from jax.experimental.pallas import tpu as pltpu
from jax.experimental.pallas import tpu_sc as plsc

N_Q = 10000
N_L = 2048
E = 320000
C = 128
AGGW = 144

NC = 2
NS = 16
NW = NC * NS
EPW = E // NW
N_QP = 10240
QPT = N_QP // NS

_MESH = plsc.VectorSubcoreMesh(core_axis_name="c", subcore_axis_name="s")
_SC_PARAMS = pltpu.CompilerParams(needs_layout_passes=False,
                                  use_tc_tiling_on_sc=False)

_SIGMA = np.concatenate(
    [32 * g + np.concatenate([2 * np.arange(16), 2 * np.arange(16) + 1])
     for g in range(4)])


CEA = 2000
NCHA = EPW // CEA


@functools.partial(
    pl.kernel,
    out_type=jax.ShapeDtypeStruct((4, E), jnp.float32),
    mesh=_MESH,
    compiler_params=_SC_PARAMS,
    scratch_types=[
        pltpu.VMEM((N_L * 2,), jnp.float32),
        pltpu.VMEM((N_Q * 2,), jnp.float32),
        pltpu.VMEM((CEA,), jnp.int32),
        pltpu.VMEM((CEA,), jnp.int32),
        pltpu.VMEM((4, CEA), jnp.float32),
    ],
)
def _gather_kin(dst_hbm, src_hbm, lat_hbm, qp_hbm, kin_hbm,
                lat_v, qp_v, src_v, dst_v, kin_v):
    wid = lax.axis_index("s") * NC + lax.axis_index("c")
    base = wid * EPW
    pltpu.sync_copy(lat_hbm, lat_v)
    pltpu.sync_copy(qp_hbm, qp_v)

    def chunk_body(ci, _):
        e0 = base + ci * CEA
        pltpu.sync_copy(src_hbm.at[pl.ds(e0, CEA)], src_v)
        pltpu.sync_copy(dst_hbm.at[pl.ds(e0, CEA)], dst_v)

        def sub(i, _):
            sv = src_v[pl.ds(i * 16, 16)] * 2
            dv = dst_v[pl.ds(i * 16, 16)] * 2
            kin_v[0, pl.ds(i * 16, 16)] = plsc.load_gather(lat_v, [sv])
            kin_v[1, pl.ds(i * 16, 16)] = plsc.load_gather(lat_v, [sv + 1])
            kin_v[2, pl.ds(i * 16, 16)] = plsc.load_gather(qp_v, [dv])
            kin_v[3, pl.ds(i * 16, 16)] = plsc.load_gather(qp_v, [dv + 1])
            return 0

        lax.fori_loop(0, CEA // 16, sub, 0)
        pltpu.sync_copy(kin_v, kin_hbm.at[:, pl.ds(e0, CEA)])
        return 0

    lax.fori_loop(0, NCHA, chunk_body, 0)


EB = 2560


def _edge_mlp_body(kin_ref, w1_ref, b1_ref, w2_ref, b2_ref, w3_ref, b3_ref,
                   k_ref):
    kin = kin_ref[...]
    h = jax.nn.gelu(
        jnp.dot(w1_ref[...], kin, preferred_element_type=jnp.float32)
        + b1_ref[...][:, None])
    h = jax.nn.gelu(
        jnp.dot(w2_ref[...], h, preferred_element_type=jnp.float32)
        + b2_ref[...][:, None])
    k = lax.dot_general(h, w3_ref[...], (((0,), (0,)), ((), ())),
                        preferred_element_type=jnp.float32)
    k_ref[...] = (k + b3_ref[...][None, :]).astype(jnp.bfloat16)


def _edge_mlp(kin_t, w1t, b1, w2t, b2, w3, b3):
    full = lambda shape: pl.BlockSpec(shape, lambda i: tuple(0 for _ in shape))
    return pl.pallas_call(
        _edge_mlp_body,
        grid=(E // EB,),
        in_specs=[
            pl.BlockSpec((4, EB), lambda i: (0, i)),
            full((64, 4)), full((64,)),
            full((64, 64)), full((64,)),
            full((64, C)), full((C,)),
        ],
        out_specs=pl.BlockSpec((EB, C), lambda i: (i, 0)),
        out_shape=jax.ShapeDtypeStruct((E, C), jnp.bfloat16),
    )(kin_t, w1t, b1, w2t, b2, w3, b3)


CE = 40
NCH = EPW // CE


@functools.partial(
    pl.kernel,
    out_type=jax.ShapeDtypeStruct((NC * N_QP, AGGW), jnp.float32),
    mesh=_MESH,
    compiler_params=_SC_PARAMS,
    scratch_types=[
        [pltpu.VMEM((CE,), jnp.int32) for _ in range(2)],
        [pltpu.VMEM((CE,), jnp.int32) for _ in range(2)],
        [pltpu.VMEM((CE, C), jnp.bfloat16) for _ in range(2)],
        [pltpu.VMEM((CE, C), jnp.bfloat16) for _ in range(2)],
        [pltpu.VMEM((CE, AGGW), jnp.float32) for _ in range(2)],
        pltpu.VMEM_SHARED((N_QP, AGGW), jnp.float32),
        [pltpu.SemaphoreType.DMA for _ in range(2)],
        [pltpu.SemaphoreType.DMA for _ in range(2)],
        [pltpu.SemaphoreType.DMA for _ in range(2)],
        [pltpu.SemaphoreType.DMA for _ in range(2)],
        [pltpu.SemaphoreType.DMA for _ in range(2)],
    ],
)
def _scatter_agg(dst_hbm, src_hbm, k_hbm, rnd_hbm, out_hbm,
                 di, si, kv, fv, mv, agg_sh, semd, semi, semk, semf, semsc):
    cid = lax.axis_index("c")
    sid = lax.axis_index("s")
    wid = sid * NC + cid
    base = wid * EPW
    z16 = jnp.zeros((16,), jnp.float32)
    cnt16 = (lax.iota(jnp.int32, 16) == 0).astype(jnp.float32)

    def zrow(i, _):
        for j in range(AGGW // 16):
            mv[0][i, pl.ds(j * 16, 16)] = z16
        return 0

    lax.fori_loop(0, CE, zrow, 0)

    def zcp(i, _):
        pltpu.sync_copy(mv[0], agg_sh.at[pl.ds(sid * QPT + i * CE, CE)])
        return 0

    lax.fori_loop(0, QPT // CE, zcp, 0)
    plsc.subcore_barrier()

    def issue_loads(ci, b):
        e0 = base + ci * CE
        pltpu.async_copy(dst_hbm.at[pl.ds(e0, CE)], di[b], semd[b])
        pltpu.async_copy(src_hbm.at[pl.ds(e0, CE)], si[b], semi[b])
        pltpu.async_copy(k_hbm.at[pl.ds(e0, CE)], kv[b], semk[b])

    issue_loads(0, 0)
    issue_loads(1, 1)
    pltpu.make_async_copy(src_hbm.at[pl.ds(base, CE)], si[0], semi[0]).wait()
    pltpu.async_copy(rnd_hbm.at[si[0]], fv[0], semf[0])

    def pair_body(cj, _):
        for b in range(2):
            ci = 2 * cj + b
            nb = 1 - b

            @pl.when(ci + 1 < NCH)
            def _():
                pltpu.make_async_copy(src_hbm.at[pl.ds(base, CE)], si[nb],
                                      semi[nb]).wait()
                pltpu.async_copy(rnd_hbm.at[si[nb]], fv[nb], semf[nb])

            pltpu.make_async_copy(k_hbm.at[pl.ds(base, CE)], kv[b],
                                  semk[b]).wait()
            pltpu.make_async_copy(rnd_hbm.at[si[b]], fv[b], semf[b]).wait()

            @pl.when(ci >= 1)
            def _():
                pltpu.make_async_copy(mv[nb], agg_sh.at[di[nb]],
                                      semsc[nb]).wait()

            @pl.when(ci + 1 < NCH)
            def _():
                e0n = base + (ci + 1) * CE
                pltpu.async_copy(dst_hbm.at[pl.ds(e0n, CE)], di[nb], semd[nb])

            pltpu.make_async_copy(dst_hbm.at[pl.ds(base, CE)], di[b],
                                  semd[b]).wait()

            def erow(e, _):
                for g in range(4):
                    ka, kb = plsc.unpack(kv[b][e, pl.ds(g * 32, 32)],
                                         format=plsc.PackFormat.INTERLEAVED)
                    fa, fb = plsc.unpack(fv[b][e, pl.ds(g * 32, 32)],
                                         format=plsc.PackFormat.INTERLEAVED)
                    mv[b][e, pl.ds(g * 32, 16)] = ka * fa
                    mv[b][e, pl.ds(g * 32 + 16, 16)] = kb * fb
                mv[b][e, pl.ds(C, 16)] = cnt16
                return 0

            lax.fori_loop(0, CE, erow, 0)

            pltpu.async_copy(mv[b], agg_sh.at[di[b]], semsc[b], add=True)

            @pl.when(ci + 2 < NCH)
            def _():
                e0nn = base + (ci + 2) * CE
                pltpu.async_copy(src_hbm.at[pl.ds(e0nn, CE)], si[b], semi[b])
                pltpu.async_copy(k_hbm.at[pl.ds(e0nn, CE)], kv[b], semk[b])
        return 0

    lax.fori_loop(0, NCH // 2, pair_body, 0)
    pltpu.make_async_copy(mv[1], agg_sh.at[di[1]], semsc[1]).wait()
    plsc.subcore_barrier()

    def rd(i, _):
        r0 = sid * QPT + i * CE
        pltpu.sync_copy(agg_sh.at[pl.ds(r0, CE)],
                        out_hbm.at[pl.ds(cid * N_QP + r0, CE)])
        return 0

    lax.fori_loop(0, QPT // CE, rd, 0)


RB = 400


def _proj_body(a0_ref, a1_ref, w1_ref, b1_ref, w2_ref, b2_ref, out_ref):
    s = a0_ref[...] + a1_ref[...]
    agg = s[:, :C]
    cnt = jnp.sum(s[:, C:], axis=1, keepdims=True)
    decoded = agg / jnp.maximum(cnt, 1.0)
    p = jax.nn.gelu(
        jnp.dot(decoded, w1_ref[...], preferred_element_type=jnp.float32)
        + b1_ref[...][None, :])
    out_ref[...] = (
        jnp.dot(p, w2_ref[...], preferred_element_type=jnp.float32)
        + b2_ref[...][None, :])


def _proj(a0, a1, w1, b1, w2, b2):
    full = lambda shape: pl.BlockSpec(shape, lambda i: tuple(0 for _ in shape))
    return pl.pallas_call(
        _proj_body,
        grid=(N_Q // RB,),
        in_specs=[
            pl.BlockSpec((RB, AGGW), lambda i: (i, 0)),
            pl.BlockSpec((RB, AGGW), lambda i: (i, 0)),
            full((C, 256)), full((256,)),
            full((256, C)), full((C,)),
        ],
        out_specs=pl.BlockSpec((RB, C), lambda i: (i, 0)),
        out_shape=jax.ShapeDtypeStruct((N_Q, C), jnp.float32),
    )(a0, a1, w1, b1, w2, b2)


def kernel(rndata_flat, phys_pos_query, batch_idx_phys_query,
           latent_tokens_pos, latent_tokens_batch_idx, edge_index,
           Wk1, bk1, Wk2, bk2, Wk3, bk3, Wp1, bp1, Wp2, bp2):
    dst = edge_index[0]
    src = edge_index[1]
    kin_t = _gather_kin(dst, src, latent_tokens_pos.reshape(-1),
                        phys_pos_query.reshape(-1))
    k = _edge_mlp(kin_t, Wk1.T, bk1, Wk2.T, bk2, Wk3, bk3)
    agg2 = _scatter_agg(dst, src, k, rndata_flat.astype(jnp.bfloat16))
    out = _proj(agg2[:N_Q], agg2[N_QP:N_QP + N_Q],
                Wp1[_SIGMA], bp1, Wp2, bp2)
    return out


# speedup vs baseline: 4.2559x; 1.0254x over previous
import functools

import jax
import jax.numpy as jnp
import numpy as np
from jax import lax
from jax.experimental import pallas as pl
from jax.experimental.pallas import tpu as pltpu
from jax.experimental.pallas import tpu_sc as plsc

N_Q = 10000
N_L = 2048
E = 320000
C = 128
AGGW = 144

NC = 2
NS = 16
NW = NC * NS
EPW = E // NW
N_QP = 10240
QPT = N_QP // NS

_MESH = plsc.VectorSubcoreMesh(core_axis_name="c", subcore_axis_name="s")
_SC_PARAMS = pltpu.CompilerParams(needs_layout_passes=False,
                                  use_tc_tiling_on_sc=False)

_SIGMA = np.concatenate(
    [32 * g + np.concatenate([2 * np.arange(16), 2 * np.arange(16) + 1])
     for g in range(4)])


CEA = 2000
NCHA = EPW // CEA


@functools.partial(
    pl.kernel,
    out_type=jax.ShapeDtypeStruct((4, E), jnp.float32),
    mesh=_MESH,
    compiler_params=_SC_PARAMS,
    scratch_types=[
        pltpu.VMEM((N_L * 2,), jnp.float32),
        pltpu.VMEM((N_Q * 2,), jnp.float32),
        pltpu.VMEM((CEA,), jnp.int32),
        pltpu.VMEM((CEA,), jnp.int32),
        pltpu.VMEM((4, CEA), jnp.float32),
    ],
)
def _gather_kin(dst_hbm, src_hbm, lat_hbm, qp_hbm, kin_hbm,
                lat_v, qp_v, src_v, dst_v, kin_v):
    wid = lax.axis_index("s") * NC + lax.axis_index("c")
    base = wid * EPW
    pltpu.sync_copy(lat_hbm, lat_v)
    pltpu.sync_copy(qp_hbm, qp_v)

    def chunk_body(ci, _):
        e0 = base + ci * CEA
        pltpu.sync_copy(src_hbm.at[pl.ds(e0, CEA)], src_v)
        pltpu.sync_copy(dst_hbm.at[pl.ds(e0, CEA)], dst_v)

        def sub(i, _):
            sv = src_v[pl.ds(i * 16, 16)] * 2
            dv = dst_v[pl.ds(i * 16, 16)] * 2
            kin_v[0, pl.ds(i * 16, 16)] = plsc.load_gather(lat_v, [sv])
            kin_v[1, pl.ds(i * 16, 16)] = plsc.load_gather(lat_v, [sv + 1])
            kin_v[2, pl.ds(i * 16, 16)] = plsc.load_gather(qp_v, [dv])
            kin_v[3, pl.ds(i * 16, 16)] = plsc.load_gather(qp_v, [dv + 1])
            return 0

        lax.fori_loop(0, CEA // 16, sub, 0)
        pltpu.sync_copy(kin_v, kin_hbm.at[:, pl.ds(e0, CEA)])
        return 0

    lax.fori_loop(0, NCHA, chunk_body, 0)


EB = 2560


def _edge_mlp_body(kin_ref, w1_ref, b1_ref, w2_ref, b2_ref, w3_ref, b3_ref,
                   k_ref):
    kin = kin_ref[...].astype(jnp.bfloat16)
    h = jax.nn.gelu(
        jnp.dot(w1_ref[...], kin, preferred_element_type=jnp.float32)
        + b1_ref[...][:, None])
    h = jax.nn.gelu(
        jnp.dot(w2_ref[...], h.astype(jnp.bfloat16),
                preferred_element_type=jnp.float32)
        + b2_ref[...][:, None])
    k = lax.dot_general(h.astype(jnp.bfloat16), w3_ref[...],
                        (((0,), (0,)), ((), ())),
                        preferred_element_type=jnp.float32)
    k_ref[...] = (k + b3_ref[...][None, :]).astype(jnp.bfloat16)


def _edge_mlp(kin_t, w1t, b1, w2t, b2, w3, b3):
    full = lambda shape: pl.BlockSpec(shape, lambda i: tuple(0 for _ in shape))
    return pl.pallas_call(
        _edge_mlp_body,
        grid=(E // EB,),
        in_specs=[
            pl.BlockSpec((4, EB), lambda i: (0, i)),
            full((64, 4)), full((64,)),
            full((64, 64)), full((64,)),
            full((64, C)), full((C,)),
        ],
        out_specs=pl.BlockSpec((EB, C), lambda i: (i, 0)),
        out_shape=jax.ShapeDtypeStruct((E, C), jnp.bfloat16),
    )(kin_t, w1t, b1, w2t, b2, w3, b3)


CE = 40
NCH = EPW // CE


@functools.partial(
    pl.kernel,
    out_type=jax.ShapeDtypeStruct((NC, N_QP, AGGW), jnp.float32),
    mesh=_MESH,
    compiler_params=_SC_PARAMS,
    scratch_types=[
        [pltpu.VMEM((CE,), jnp.int32) for _ in range(2)],
        [pltpu.VMEM((CE,), jnp.int32) for _ in range(2)],
        [pltpu.VMEM((CE, C), jnp.bfloat16) for _ in range(2)],
        [pltpu.VMEM((CE, C), jnp.bfloat16) for _ in range(2)],
        [pltpu.VMEM((CE, AGGW), jnp.float32) for _ in range(2)],
        pltpu.VMEM_SHARED((N_QP, AGGW), jnp.float32),
        [pltpu.SemaphoreType.DMA for _ in range(2)],
        [pltpu.SemaphoreType.DMA for _ in range(2)],
        [pltpu.SemaphoreType.DMA for _ in range(2)],
        [pltpu.SemaphoreType.DMA for _ in range(2)],
        [pltpu.SemaphoreType.DMA for _ in range(2)],
    ],
)
def _scatter_agg(dst_hbm, src_hbm, k_hbm, rnd_hbm, out_hbm,
                 di, si, kv, fv, mv, agg_sh, semd, semi, semk, semf, semsc):
    cid = lax.axis_index("c")
    sid = lax.axis_index("s")
    wid = sid * NC + cid
    base = wid * EPW
    z16 = jnp.zeros((16,), jnp.float32)
    cnt16 = (lax.iota(jnp.int32, 16) == 0).astype(jnp.float32)

    def zrow(i, _):
        for j in range(AGGW // 16):
            mv[0][i, pl.ds(j * 16, 16)] = z16
        return 0

    lax.fori_loop(0, CE, zrow, 0)

    def zcp(i, _):
        pltpu.sync_copy(mv[0], agg_sh.at[pl.ds(sid * QPT + i * CE, CE)])
        return 0

    lax.fori_loop(0, QPT // CE, zcp, 0)
    plsc.subcore_barrier()

    def issue_loads(ci, b):
        e0 = base + ci * CE
        pltpu.async_copy(dst_hbm.at[pl.ds(e0, CE)], di[b], semd[b])
        pltpu.async_copy(src_hbm.at[pl.ds(e0, CE)], si[b], semi[b])
        pltpu.async_copy(k_hbm.at[pl.ds(e0, CE)], kv[b], semk[b])

    issue_loads(0, 0)
    issue_loads(1, 1)
    pltpu.make_async_copy(src_hbm.at[pl.ds(base, CE)], si[0], semi[0]).wait()
    pltpu.async_copy(rnd_hbm.at[si[0]], fv[0], semf[0])

    def pair_body(cj, _):
        for b in range(2):
            ci = 2 * cj + b
            nb = 1 - b

            @pl.when(ci + 1 < NCH)
            def _():
                pltpu.make_async_copy(src_hbm.at[pl.ds(base, CE)], si[nb],
                                      semi[nb]).wait()
                pltpu.async_copy(rnd_hbm.at[si[nb]], fv[nb], semf[nb])

            pltpu.make_async_copy(k_hbm.at[pl.ds(base, CE)], kv[b],
                                  semk[b]).wait()
            pltpu.make_async_copy(rnd_hbm.at[si[b]], fv[b], semf[b]).wait()

            @pl.when(ci >= 1)
            def _():
                pltpu.make_async_copy(mv[nb], agg_sh.at[di[nb]],
                                      semsc[nb]).wait()

            @pl.when(ci + 1 < NCH)
            def _():
                e0n = base + (ci + 1) * CE
                pltpu.async_copy(dst_hbm.at[pl.ds(e0n, CE)], di[nb], semd[nb])

            pltpu.make_async_copy(dst_hbm.at[pl.ds(base, CE)], di[b],
                                  semd[b]).wait()

            def erow(e, _):
                for g in range(4):
                    ka, kb = plsc.unpack(kv[b][e, pl.ds(g * 32, 32)],
                                         format=plsc.PackFormat.INTERLEAVED)
                    fa, fb = plsc.unpack(fv[b][e, pl.ds(g * 32, 32)],
                                         format=plsc.PackFormat.INTERLEAVED)
                    mv[b][e, pl.ds(g * 32, 16)] = ka * fa
                    mv[b][e, pl.ds(g * 32 + 16, 16)] = kb * fb
                mv[b][e, pl.ds(C, 16)] = cnt16
                return 0

            lax.fori_loop(0, CE, erow, 0)

            pltpu.async_copy(mv[b], agg_sh.at[di[b]], semsc[b], add=True)

            @pl.when(ci + 2 < NCH)
            def _():
                e0nn = base + (ci + 2) * CE
                pltpu.async_copy(src_hbm.at[pl.ds(e0nn, CE)], si[b], semi[b])
                pltpu.async_copy(k_hbm.at[pl.ds(e0nn, CE)], kv[b], semk[b])
        return 0

    lax.fori_loop(0, NCH // 2, pair_body, 0)
    pltpu.make_async_copy(mv[1], agg_sh.at[di[1]], semsc[1]).wait()
    plsc.subcore_barrier()

    def rd(i, _):
        r0 = sid * QPT + i * CE
        pltpu.sync_copy(agg_sh.at[pl.ds(r0, CE)],
                        out_hbm.at[cid, pl.ds(r0, CE)])
        return 0

    lax.fori_loop(0, QPT // CE, rd, 0)


RB = 400


def _proj_body(a0_ref, a1_ref, w1_ref, b1_ref, w2_ref, b2_ref, out_ref):
    s = a0_ref[0] + a1_ref[0]
    agg = s[:, :C]
    cnt = jnp.sum(s[:, C:], axis=1, keepdims=True)
    decoded = agg / jnp.maximum(cnt, 1.0)
    p = jax.nn.gelu(
        jnp.dot(decoded, w1_ref[...], preferred_element_type=jnp.float32)
        + b1_ref[...][None, :])
    out_ref[...] = (
        jnp.dot(p, w2_ref[...], preferred_element_type=jnp.float32)
        + b2_ref[...][None, :])


def _proj(a0, a1, w1, b1, w2, b2):
    full = lambda shape: pl.BlockSpec(shape, lambda i: tuple(0 for _ in shape))
    return pl.pallas_call(
        _proj_body,
        grid=(N_Q // RB,),
        in_specs=[
            pl.BlockSpec((1, RB, AGGW), lambda i: (0, i, 0)),
            pl.BlockSpec((1, RB, AGGW), lambda i: (1, i, 0)),
            full((C, 256)), full((256,)),
            full((256, C)), full((C,)),
        ],
        out_specs=pl.BlockSpec((RB, C), lambda i: (i, 0)),
        out_shape=jax.ShapeDtypeStruct((N_Q, C), jnp.float32),
    )(a0, a1, w1, b1, w2, b2)


def kernel(rndata_flat, phys_pos_query, batch_idx_phys_query,
           latent_tokens_pos, latent_tokens_batch_idx, edge_index,
           Wk1, bk1, Wk2, bk2, Wk3, bk3, Wp1, bp1, Wp2, bp2):
    dst = edge_index[0]
    src = edge_index[1]
    kin_t = _gather_kin(dst, src, latent_tokens_pos.reshape(-1),
                        phys_pos_query.reshape(-1))
    k = _edge_mlp(kin_t, Wk1.T.astype(jnp.bfloat16), bk1,
                  Wk2.T.astype(jnp.bfloat16), bk2,
                  Wk3.astype(jnp.bfloat16), bk3)
    agg2 = _scatter_agg(dst, src, k, rndata_flat.astype(jnp.bfloat16))
    out = _proj(agg2, agg2, Wp1[_SIGMA], bp1, Wp2, bp2)
    return out


# speedup vs baseline: 4.2796x; 1.0056x over previous
import functools

import jax
import jax.numpy as jnp
import numpy as np
from jax import lax
from jax.experimental import pallas as pl
from jax.experimental.pallas import tpu as pltpu
from jax.experimental.pallas import tpu_sc as plsc

N_Q = 10000
N_L = 2048
E = 320000
C = 128
AGGW = 144

NC = 2
NS = 16
NW = NC * NS
EPW = E // NW
N_QP = 10240
QPT = N_QP // NS

_MESH = plsc.VectorSubcoreMesh(core_axis_name="c", subcore_axis_name="s")
_SC_PARAMS = pltpu.CompilerParams(needs_layout_passes=False,
                                  use_tc_tiling_on_sc=False)

_SIGMA = np.concatenate(
    [32 * g + np.concatenate([2 * np.arange(16), 2 * np.arange(16) + 1])
     for g in range(4)])


CEA = 2000
NCHA = EPW // CEA


@functools.partial(
    pl.kernel,
    out_type=jax.ShapeDtypeStruct((4, E), jnp.float32),
    mesh=_MESH,
    compiler_params=_SC_PARAMS,
    scratch_types=[
        pltpu.VMEM((N_L * 2,), jnp.float32),
        pltpu.VMEM((N_Q * 2,), jnp.float32),
        pltpu.VMEM((CEA,), jnp.int32),
        pltpu.VMEM((CEA,), jnp.int32),
        pltpu.VMEM((4, CEA), jnp.float32),
    ],
)
def _gather_kin(dst_hbm, src_hbm, lat_hbm, qp_hbm, kin_hbm,
                lat_v, qp_v, src_v, dst_v, kin_v):
    wid = lax.axis_index("s") * NC + lax.axis_index("c")
    base = wid * EPW
    pltpu.sync_copy(lat_hbm, lat_v)
    pltpu.sync_copy(qp_hbm, qp_v)

    def chunk_body(ci, _):
        e0 = base + ci * CEA
        pltpu.sync_copy(src_hbm.at[pl.ds(e0, CEA)], src_v)
        pltpu.sync_copy(dst_hbm.at[pl.ds(e0, CEA)], dst_v)

        def sub(i, _):
            sv = src_v[pl.ds(i * 16, 16)] * 2
            dv = dst_v[pl.ds(i * 16, 16)] * 2
            kin_v[0, pl.ds(i * 16, 16)] = plsc.load_gather(lat_v, [sv])
            kin_v[1, pl.ds(i * 16, 16)] = plsc.load_gather(lat_v, [sv + 1])
            kin_v[2, pl.ds(i * 16, 16)] = plsc.load_gather(qp_v, [dv])
            kin_v[3, pl.ds(i * 16, 16)] = plsc.load_gather(qp_v, [dv + 1])
            return 0

        lax.fori_loop(0, CEA // 16, sub, 0)
        pltpu.sync_copy(kin_v, kin_hbm.at[:, pl.ds(e0, CEA)])
        return 0

    lax.fori_loop(0, NCHA, chunk_body, 0)


EB = 2560


def _edge_mlp_body(kin_ref, w1_ref, b1_ref, w2_ref, b2_ref, w3_ref, b3_ref,
                   k_ref):
    kin = kin_ref[...].astype(jnp.bfloat16)
    h = jax.nn.gelu(
        jnp.dot(w1_ref[...], kin, preferred_element_type=jnp.float32)
        + b1_ref[...][:, None])
    h = jax.nn.gelu(
        jnp.dot(w2_ref[...], h.astype(jnp.bfloat16),
                preferred_element_type=jnp.float32)
        + b2_ref[...][:, None])
    k = lax.dot_general(h.astype(jnp.bfloat16), w3_ref[...],
                        (((0,), (0,)), ((), ())),
                        preferred_element_type=jnp.float32)
    k_ref[...] = (k + b3_ref[...][None, :]).astype(jnp.bfloat16)


def _edge_mlp(kin_t, w1t, b1, w2t, b2, w3, b3):
    full = lambda shape: pl.BlockSpec(shape, lambda i: tuple(0 for _ in shape))
    return pl.pallas_call(
        _edge_mlp_body,
        grid=(E // EB,),
        in_specs=[
            pl.BlockSpec((4, EB), lambda i: (0, i)),
            full((64, 4)), full((64,)),
            full((64, 64)), full((64,)),
            full((64, C)), full((C,)),
        ],
        out_specs=pl.BlockSpec((EB, C), lambda i: (i, 0)),
        out_shape=jax.ShapeDtypeStruct((E, C), jnp.bfloat16),
    )(kin_t, w1t, b1, w2t, b2, w3, b3)


CE = 40
NCH = EPW // CE


@functools.partial(
    pl.kernel,
    out_type=(jax.ShapeDtypeStruct((NC, N_QP, C), jnp.float32),
              jax.ShapeDtypeStruct((NC, N_QP, 16), jnp.float32)),
    mesh=_MESH,
    compiler_params=_SC_PARAMS,
    scratch_types=[
        [pltpu.VMEM((CE,), jnp.int32) for _ in range(2)],
        [pltpu.VMEM((CE,), jnp.int32) for _ in range(2)],
        [pltpu.VMEM((CE, C), jnp.bfloat16) for _ in range(2)],
        [pltpu.VMEM((CE, C), jnp.bfloat16) for _ in range(2)],
        [pltpu.VMEM((CE, AGGW), jnp.float32) for _ in range(2)],
        pltpu.VMEM_SHARED((N_QP, AGGW), jnp.float32),
        [pltpu.SemaphoreType.DMA for _ in range(2)],
        [pltpu.SemaphoreType.DMA for _ in range(2)],
        [pltpu.SemaphoreType.DMA for _ in range(2)],
        [pltpu.SemaphoreType.DMA for _ in range(2)],
        [pltpu.SemaphoreType.DMA for _ in range(2)],
    ],
)
def _scatter_agg(dst_hbm, src_hbm, k_hbm, rnd_hbm, outm_hbm, outc_hbm,
                 di, si, kv, fv, mv, agg_sh, semd, semi, semk, semf, semsc):
    cid = lax.axis_index("c")
    sid = lax.axis_index("s")
    wid = sid * NC + cid
    base = wid * EPW
    z16 = jnp.zeros((16,), jnp.float32)
    cnt16 = (lax.iota(jnp.int32, 16) == 0).astype(jnp.float32)

    def zrow(i, _):
        for j in range(AGGW // 16):
            mv[0][i, pl.ds(j * 16, 16)] = z16
        return 0

    lax.fori_loop(0, CE, zrow, 0)

    def zcp(i, _):
        pltpu.sync_copy(mv[0], agg_sh.at[pl.ds(sid * QPT + i * CE, CE)])
        return 0

    lax.fori_loop(0, QPT // CE, zcp, 0)

    def crow(i, _):
        mv[0][i, pl.ds(C, 16)] = cnt16
        mv[1][i, pl.ds(C, 16)] = cnt16
        return 0

    lax.fori_loop(0, CE, crow, 0)
    plsc.subcore_barrier()

    def issue_loads(ci, b):
        e0 = base + ci * CE
        pltpu.async_copy(dst_hbm.at[pl.ds(e0, CE)], di[b], semd[b])
        pltpu.async_copy(src_hbm.at[pl.ds(e0, CE)], si[b], semi[b])
        pltpu.async_copy(k_hbm.at[pl.ds(e0, CE)], kv[b], semk[b])

    issue_loads(0, 0)
    issue_loads(1, 1)
    pltpu.make_async_copy(src_hbm.at[pl.ds(base, CE)], si[0], semi[0]).wait()
    pltpu.async_copy(rnd_hbm.at[si[0]], fv[0], semf[0])

    def pair_body(cj, _):
        for b in range(2):
            ci = 2 * cj + b
            nb = 1 - b

            @pl.when(ci + 1 < NCH)
            def _():
                pltpu.make_async_copy(src_hbm.at[pl.ds(base, CE)], si[nb],
                                      semi[nb]).wait()
                pltpu.async_copy(rnd_hbm.at[si[nb]], fv[nb], semf[nb])

            pltpu.make_async_copy(k_hbm.at[pl.ds(base, CE)], kv[b],
                                  semk[b]).wait()
            pltpu.make_async_copy(rnd_hbm.at[si[b]], fv[b], semf[b]).wait()

            @pl.when(ci >= 1)
            def _():
                pltpu.make_async_copy(mv[nb], agg_sh.at[di[nb]],
                                      semsc[nb]).wait()

            @pl.when(ci + 1 < NCH)
            def _():
                e0n = base + (ci + 1) * CE
                pltpu.async_copy(dst_hbm.at[pl.ds(e0n, CE)], di[nb], semd[nb])

            pltpu.make_async_copy(dst_hbm.at[pl.ds(base, CE)], di[b],
                                  semd[b]).wait()

            def erow(e, _):
                for g in range(4):
                    prod = (kv[b][e, pl.ds(g * 32, 32)]
                            * fv[b][e, pl.ds(g * 32, 32)])
                    pa, pb = plsc.unpack(prod,
                                         format=plsc.PackFormat.INTERLEAVED)
                    mv[b][e, pl.ds(g * 32, 16)] = pa
                    mv[b][e, pl.ds(g * 32 + 16, 16)] = pb
                return 0

            lax.fori_loop(0, CE, erow, 0)

            pltpu.async_copy(mv[b], agg_sh.at[di[b]], semsc[b], add=True)

            @pl.when(ci + 2 < NCH)
            def _():
                e0nn = base + (ci + 2) * CE
                pltpu.async_copy(src_hbm.at[pl.ds(e0nn, CE)], si[b], semi[b])
                pltpu.async_copy(k_hbm.at[pl.ds(e0nn, CE)], kv[b], semk[b])
        return 0

    lax.fori_loop(0, NCH // 2, pair_body, 0)
    pltpu.make_async_copy(mv[1], agg_sh.at[di[1]], semsc[1]).wait()
    plsc.subcore_barrier()

    def rd(i, _):
        r0 = sid * QPT + i * CE
        pltpu.sync_copy(agg_sh.at[pl.ds(r0, CE), pl.ds(0, C)],
                        outm_hbm.at[cid, pl.ds(r0, CE)])
        pltpu.sync_copy(agg_sh.at[pl.ds(r0, CE), pl.ds(C, 16)],
                        outc_hbm.at[cid, pl.ds(r0, CE)])
        return 0

    lax.fori_loop(0, QPT // CE, rd, 0)


RB = 400


def _proj_body(m0_ref, m1_ref, c0_ref, c1_ref, w1_ref, b1_ref, w2_ref,
               b2_ref, out_ref):
    agg = m0_ref[0] + m1_ref[0]
    cnt = jnp.sum(c0_ref[0] + c1_ref[0], axis=1, keepdims=True)
    decoded = agg / jnp.maximum(cnt, 1.0)
    p = jax.nn.gelu(
        jnp.dot(decoded, w1_ref[...], preferred_element_type=jnp.float32)
        + b1_ref[...][None, :])
    out_ref[...] = (
        jnp.dot(p, w2_ref[...], preferred_element_type=jnp.float32)
        + b2_ref[...][None, :])


def _proj(m, c, w1, b1, w2, b2):
    full = lambda shape: pl.BlockSpec(shape, lambda i: tuple(0 for _ in shape))
    return pl.pallas_call(
        _proj_body,
        grid=(N_Q // RB,),
        in_specs=[
            pl.BlockSpec((1, RB, C), lambda i: (0, i, 0)),
            pl.BlockSpec((1, RB, C), lambda i: (1, i, 0)),
            pl.BlockSpec((1, RB, 16), lambda i: (0, i, 0)),
            pl.BlockSpec((1, RB, 16), lambda i: (1, i, 0)),
            full((C, 256)), full((256,)),
            full((256, C)), full((C,)),
        ],
        out_specs=pl.BlockSpec((RB, C), lambda i: (i, 0)),
        out_shape=jax.ShapeDtypeStruct((N_Q, C), jnp.float32),
    )(m, m, c, c, w1, b1, w2, b2)


def kernel(rndata_flat, phys_pos_query, batch_idx_phys_query,
           latent_tokens_pos, latent_tokens_batch_idx, edge_index,
           Wk1, bk1, Wk2, bk2, Wk3, bk3, Wp1, bp1, Wp2, bp2):
    dst = edge_index[0]
    src = edge_index[1]
    kin_t = _gather_kin(dst, src, latent_tokens_pos.reshape(-1),
                        phys_pos_query.reshape(-1))
    k = _edge_mlp(kin_t, Wk1.T.astype(jnp.bfloat16), bk1,
                  Wk2.T.astype(jnp.bfloat16), bk2,
                  Wk3.astype(jnp.bfloat16), bk3)
    aggm, aggc = _scatter_agg(dst, src, k, rndata_flat.astype(jnp.bfloat16))
    out = _proj(aggm, aggc, Wp1[_SIGMA], bp1, Wp2, bp2)
    return out


# speedup vs baseline: 4.9082x; 1.1469x over previous
import functools

import jax
import jax.numpy as jnp
import numpy as np
from jax import lax
from jax.experimental import pallas as pl
from jax.experimental.pallas import tpu as pltpu
from jax.experimental.pallas import tpu_sc as plsc

N_Q = 10000
N_L = 2048
E = 320000
C = 128
AGGW = 144

NC = 2
NS = 16
NW = NC * NS
EPW = E // NW
N_QP = 10240
QPT = N_QP // NS

_MESH = plsc.VectorSubcoreMesh(core_axis_name="c", subcore_axis_name="s")
_SC_PARAMS = pltpu.CompilerParams(needs_layout_passes=False,
                                  use_tc_tiling_on_sc=False)

_SIGMA = np.concatenate(
    [32 * g + np.concatenate([2 * np.arange(16), 2 * np.arange(16) + 1])
     for g in range(4)])


CEA = 2000
NCHA = EPW // CEA


@functools.partial(
    pl.kernel,
    out_type=jax.ShapeDtypeStruct((4, E), jnp.float32),
    mesh=_MESH,
    compiler_params=_SC_PARAMS,
    scratch_types=[
        pltpu.VMEM((N_L * 2,), jnp.float32),
        pltpu.VMEM((N_Q * 2,), jnp.float32),
        pltpu.VMEM((2, CEA), jnp.int32),
        pltpu.VMEM((4, CEA), jnp.float32),
    ],
)
def _gather_kin(ei_hbm, lat_hbm, qp_hbm, kin_hbm, lat_v, qp_v, ei_v, kin_v):
    wid = lax.axis_index("s") * NC + lax.axis_index("c")
    base = wid * EPW
    pltpu.sync_copy(lat_hbm, lat_v)
    pltpu.sync_copy(qp_hbm, qp_v)

    def chunk_body(ci, _):
        e0 = base + ci * CEA
        pltpu.sync_copy(ei_hbm.at[:, pl.ds(e0, CEA)], ei_v)

        def sub(i, _):
            sv = ei_v[1, pl.ds(i * 16, 16)] * 2
            dv = ei_v[0, pl.ds(i * 16, 16)] * 2
            kin_v[0, pl.ds(i * 16, 16)] = plsc.load_gather(lat_v, [sv])
            kin_v[1, pl.ds(i * 16, 16)] = plsc.load_gather(lat_v, [sv + 1])
            kin_v[2, pl.ds(i * 16, 16)] = plsc.load_gather(qp_v, [dv])
            kin_v[3, pl.ds(i * 16, 16)] = plsc.load_gather(qp_v, [dv + 1])
            return 0

        lax.fori_loop(0, CEA // 16, sub, 0)
        pltpu.sync_copy(kin_v, kin_hbm.at[:, pl.ds(e0, CEA)])
        return 0

    lax.fori_loop(0, NCHA, chunk_body, 0)


EB = 2560


def _edge_mlp_body(kin_ref, w1_ref, b1_ref, w2_ref, b2_ref, w3_ref, b3_ref,
                   k_ref):
    kin = kin_ref[...].astype(jnp.bfloat16)
    h = jax.nn.gelu(
        jnp.dot(w1_ref[...], kin, preferred_element_type=jnp.float32)
        + b1_ref[...][:, None])
    h = jax.nn.gelu(
        jnp.dot(w2_ref[...], h.astype(jnp.bfloat16),
                preferred_element_type=jnp.float32)
        + b2_ref[...][:, None])
    k = lax.dot_general(h.astype(jnp.bfloat16), w3_ref[...],
                        (((0,), (0,)), ((), ())),
                        preferred_element_type=jnp.float32)
    k_ref[...] = (k + b3_ref[...][None, :]).astype(jnp.bfloat16)


def _edge_mlp(kin_t, w1t, b1, w2t, b2, w3, b3):
    full = lambda shape: pl.BlockSpec(shape, lambda i: tuple(0 for _ in shape))
    return pl.pallas_call(
        _edge_mlp_body,
        grid=(E // EB,),
        in_specs=[
            pl.BlockSpec((4, EB), lambda i: (0, i)),
            full((64, 4)), full((64,)),
            full((64, 64)), full((64,)),
            full((64, C)), full((C,)),
        ],
        out_specs=pl.BlockSpec((EB, C), lambda i: (i, 0)),
        out_shape=jax.ShapeDtypeStruct((E, C), jnp.bfloat16),
    )(kin_t, w1t, b1, w2t, b2, w3, b3)


CE = 40
NCH = EPW // CE


@functools.partial(
    pl.kernel,
    out_type=(jax.ShapeDtypeStruct((NC, N_QP, C), jnp.float32),
              jax.ShapeDtypeStruct((NC, N_QP, 16), jnp.float32)),
    mesh=_MESH,
    compiler_params=_SC_PARAMS,
    scratch_types=[
        [pltpu.VMEM((2, CE), jnp.int32) for _ in range(4)],
        [pltpu.VMEM((CE, C), jnp.bfloat16) for _ in range(4)],
        [pltpu.VMEM((CE, C), jnp.bfloat16) for _ in range(4)],
        [pltpu.VMEM((CE, AGGW), jnp.float32) for _ in range(2)],
        pltpu.VMEM_SHARED((N_QP, AGGW), jnp.float32),
        [pltpu.SemaphoreType.DMA for _ in range(4)],
        [pltpu.SemaphoreType.DMA for _ in range(4)],
        [pltpu.SemaphoreType.DMA for _ in range(4)],
        [pltpu.SemaphoreType.DMA for _ in range(2)],
    ],
)
def _scatter_agg(ei_hbm, k_hbm, rnd_hbm, outm_hbm, outc_hbm,
                 ib, kv, fv, mv, agg_sh, semd, semk, semf, semsc):
    cid = lax.axis_index("c")
    sid = lax.axis_index("s")
    wid = sid * NC + cid
    base = wid * EPW
    z16 = jnp.zeros((16,), jnp.float32)
    cnt16 = (lax.iota(jnp.int32, 16) == 0).astype(jnp.float32)

    def zrow(i, _):
        for j in range(AGGW // 16):
            mv[0][i, pl.ds(j * 16, 16)] = z16
        return 0

    lax.fori_loop(0, CE, zrow, 0)

    def zcp(i, _):
        pltpu.sync_copy(mv[0], agg_sh.at[pl.ds(sid * QPT + i * CE, CE)])
        return 0

    lax.fori_loop(0, QPT // CE, zcp, 0)

    def crow(i, _):
        mv[0][i, pl.ds(C, 16)] = cnt16
        mv[1][i, pl.ds(C, 16)] = cnt16
        return 0

    lax.fori_loop(0, CE, crow, 0)
    plsc.subcore_barrier()

    def issue_ib(ci, r):
        pltpu.async_copy(ei_hbm.at[:, pl.ds(base + ci * CE, CE)], ib[r],
                         semd[r])

    def issue_k(ci, r):
        pltpu.async_copy(k_hbm.at[pl.ds(base + ci * CE, CE)], kv[r], semk[r])

    def issue_f(r):
        pltpu.async_copy(rnd_hbm.at[ib[r].at[1]], fv[r], semf[r])

    def wait_d(r):
        pltpu.make_async_copy(ei_hbm.at[:, pl.ds(base, CE)], ib[r],
                              semd[r]).wait()

    def wait_k(r):
        pltpu.make_async_copy(k_hbm.at[pl.ds(base, CE)], kv[r], semk[r]).wait()

    def wait_f(r):
        pltpu.make_async_copy(rnd_hbm.at[ib[r].at[1]], fv[r], semf[r]).wait()

    def compute(r, m):
        def erow(e, _):
            for g in range(4):
                prod = kv[r][e, pl.ds(g * 32, 32)] * fv[r][e, pl.ds(g * 32, 32)]
                pa, pb = plsc.unpack(prod, format=plsc.PackFormat.INTERLEAVED)
                mv[m][e, pl.ds(g * 32, 16)] = pa
                mv[m][e, pl.ds(g * 32 + 16, 16)] = pb
            return 0

        lax.fori_loop(0, CE, erow, 0)

    def issue_sc(r, m):
        pltpu.async_copy(mv[m], agg_sh.at[ib[r].at[0]], semsc[m], add=True)

    def wait_sc(m):
        pltpu.make_async_copy(mv[m], agg_sh.at[ib[0].at[0]], semsc[m]).wait()

    issue_ib(0, 0)
    issue_ib(1, 1)
    issue_k(0, 0)
    issue_k(1, 1)
    wait_d(0)
    issue_f(0)
    wait_d(1)
    issue_f(1)
    wait_k(0)
    wait_f(0)
    issue_ib(2, 2)
    issue_k(2, 2)
    compute(0, 0)
    issue_sc(0, 0)
    wait_d(2)
    issue_f(2)
    wait_k(1)
    wait_f(1)
    issue_ib(3, 3)
    issue_k(3, 3)
    compute(1, 1)
    issue_sc(1, 1)

    def quad(cj, _):
        for u in range(4):
            ci = 2 + 4 * cj + u
            r = (2 + u) % 4
            nr = (3 + u) % 4
            r2 = u
            m = u % 2

            @pl.when(ci + 1 < NCH)
            def _():
                wait_d(nr)
                issue_f(nr)

            wait_k(r)
            wait_f(r)
            wait_sc(m)

            @pl.when(ci + 2 < NCH)
            def _():
                issue_ib(ci + 2, r2)
                issue_k(ci + 2, r2)

            compute(r, m)
            issue_sc(r, m)
        return 0

    lax.fori_loop(0, (NCH - 2) // 4, quad, 0)
    wait_sc(0)
    wait_sc(1)
    plsc.subcore_barrier()

    def rd(i, _):
        r0 = sid * QPT + i * CE
        pltpu.sync_copy(agg_sh.at[pl.ds(r0, CE), pl.ds(0, C)],
                        outm_hbm.at[cid, pl.ds(r0, CE)])
        pltpu.sync_copy(agg_sh.at[pl.ds(r0, CE), pl.ds(C, 16)],
                        outc_hbm.at[cid, pl.ds(r0, CE)])
        return 0

    lax.fori_loop(0, QPT // CE, rd, 0)


RB = 400


def _proj_body(m0_ref, m1_ref, c0_ref, c1_ref, w1_ref, b1_ref, w2_ref,
               b2_ref, out_ref):
    agg = m0_ref[0] + m1_ref[0]
    cnt = jnp.sum(c0_ref[0] + c1_ref[0], axis=1, keepdims=True)
    decoded = agg / jnp.maximum(cnt, 1.0)
    p = jax.nn.gelu(
        jnp.dot(decoded, w1_ref[...], preferred_element_type=jnp.float32)
        + b1_ref[...][None, :])
    out_ref[...] = (
        jnp.dot(p, w2_ref[...], preferred_element_type=jnp.float32)
        + b2_ref[...][None, :])


def _proj(m, c, w1, b1, w2, b2):
    full = lambda shape: pl.BlockSpec(shape, lambda i: tuple(0 for _ in shape))
    return pl.pallas_call(
        _proj_body,
        grid=(N_Q // RB,),
        in_specs=[
            pl.BlockSpec((1, RB, C), lambda i: (0, i, 0)),
            pl.BlockSpec((1, RB, C), lambda i: (1, i, 0)),
            pl.BlockSpec((1, RB, 16), lambda i: (0, i, 0)),
            pl.BlockSpec((1, RB, 16), lambda i: (1, i, 0)),
            full((C, 256)), full((256,)),
            full((256, C)), full((C,)),
        ],
        out_specs=pl.BlockSpec((RB, C), lambda i: (i, 0)),
        out_shape=jax.ShapeDtypeStruct((N_Q, C), jnp.float32),
    )(m, m, c, c, w1, b1, w2, b2)


def kernel(rndata_flat, phys_pos_query, batch_idx_phys_query,
           latent_tokens_pos, latent_tokens_batch_idx, edge_index,
           Wk1, bk1, Wk2, bk2, Wk3, bk3, Wp1, bp1, Wp2, bp2):
    kin_t = _gather_kin(edge_index, latent_tokens_pos.reshape(-1),
                        phys_pos_query.reshape(-1))
    k = _edge_mlp(kin_t, Wk1.T.astype(jnp.bfloat16), bk1,
                  Wk2.T.astype(jnp.bfloat16), bk2,
                  Wk3.astype(jnp.bfloat16), bk3)
    aggm, aggc = _scatter_agg(edge_index, k,
                              rndata_flat.astype(jnp.bfloat16))
    out = _proj(aggm, aggc, Wp1[_SIGMA], bp1, Wp2, bp2)
    return out


# speedup vs baseline: 4.9289x; 1.0042x over previous
import functools

import jax
import jax.numpy as jnp
import numpy as np
from jax import lax
from jax.experimental import pallas as pl
from jax.experimental.pallas import tpu as pltpu
from jax.experimental.pallas import tpu_sc as plsc

N_Q = 10000
N_L = 2048
E = 320000
C = 128
AGGW = 144

NC = 2
NS = 16
NW = NC * NS
EPW = E // NW
N_QP = 10240
QPT = N_QP // NS

_MESH = plsc.VectorSubcoreMesh(core_axis_name="c", subcore_axis_name="s")
_SC_PARAMS = pltpu.CompilerParams(needs_layout_passes=False,
                                  use_tc_tiling_on_sc=False)

_SIGMA = np.concatenate(
    [32 * g + np.concatenate([2 * np.arange(16), 2 * np.arange(16) + 1])
     for g in range(4)])


CEA = 2000
NCHA = EPW // CEA


@functools.partial(
    pl.kernel,
    out_type=jax.ShapeDtypeStruct((4, E), jnp.float32),
    mesh=_MESH,
    compiler_params=_SC_PARAMS,
    scratch_types=[
        pltpu.VMEM((N_L * 2,), jnp.float32),
        pltpu.VMEM((N_Q * 2,), jnp.float32),
        pltpu.VMEM((2, CEA), jnp.int32),
        pltpu.VMEM((4, CEA), jnp.float32),
    ],
)
def _gather_kin(ei_hbm, lat_hbm, qp_hbm, kin_hbm, lat_v, qp_v, ei_v, kin_v):
    wid = lax.axis_index("s") * NC + lax.axis_index("c")
    base = wid * EPW
    pltpu.sync_copy(lat_hbm, lat_v)
    pltpu.sync_copy(qp_hbm, qp_v)

    def chunk_body(ci, _):
        e0 = base + ci * CEA
        pltpu.sync_copy(ei_hbm.at[:, pl.ds(e0, CEA)], ei_v)

        def sub(i, _):
            sv = ei_v[1, pl.ds(i * 16, 16)] * 2
            dv = ei_v[0, pl.ds(i * 16, 16)] * 2
            kin_v[0, pl.ds(i * 16, 16)] = plsc.load_gather(lat_v, [sv])
            kin_v[1, pl.ds(i * 16, 16)] = plsc.load_gather(lat_v, [sv + 1])
            kin_v[2, pl.ds(i * 16, 16)] = plsc.load_gather(qp_v, [dv])
            kin_v[3, pl.ds(i * 16, 16)] = plsc.load_gather(qp_v, [dv + 1])
            return 0

        lax.fori_loop(0, CEA // 16, sub, 0)
        pltpu.sync_copy(kin_v, kin_hbm.at[:, pl.ds(e0, CEA)])
        return 0

    lax.fori_loop(0, NCHA, chunk_body, 0)


EB = 2560


def _edge_mlp_body(kin_ref, w1_ref, b1_ref, w2_ref, b2_ref, w3_ref, b3_ref,
                   k_ref):
    kin = kin_ref[...].astype(jnp.bfloat16)
    h = jax.nn.gelu(
        jnp.dot(w1_ref[...], kin, preferred_element_type=jnp.float32)
        + b1_ref[...][:, None])
    h = jax.nn.gelu(
        jnp.dot(w2_ref[...], h.astype(jnp.bfloat16),
                preferred_element_type=jnp.float32)
        + b2_ref[...][:, None])
    k = lax.dot_general(h.astype(jnp.bfloat16), w3_ref[...],
                        (((0,), (0,)), ((), ())),
                        preferred_element_type=jnp.float32)
    k_ref[...] = (k + b3_ref[...][None, :]).astype(jnp.bfloat16)


def _edge_mlp(kin_t, w1t, b1, w2t, b2, w3, b3):
    full = lambda shape: pl.BlockSpec(shape, lambda i: tuple(0 for _ in shape))
    return pl.pallas_call(
        _edge_mlp_body,
        grid=(E // EB,),
        in_specs=[
            pl.BlockSpec((4, EB), lambda i: (0, i)),
            full((64, 4)), full((64,)),
            full((64, 64)), full((64,)),
            full((64, C)), full((C,)),
        ],
        out_specs=pl.BlockSpec((EB, C), lambda i: (i, 0)),
        out_shape=jax.ShapeDtypeStruct((E, C), jnp.bfloat16),
    )(kin_t, w1t, b1, w2t, b2, w3, b3)


CE = 40
NCH = EPW // CE


@functools.partial(
    pl.kernel,
    out_type=(jax.ShapeDtypeStruct((NC, N_QP, C), jnp.float32),
              jax.ShapeDtypeStruct((NC, N_QP, 16), jnp.float32)),
    mesh=_MESH,
    compiler_params=_SC_PARAMS,
    scratch_types=[
        [pltpu.VMEM((2, CE), jnp.int32) for _ in range(4)],
        [pltpu.VMEM((CE, C), jnp.bfloat16) for _ in range(4)],
        [pltpu.VMEM((CE, C), jnp.bfloat16) for _ in range(4)],
        [pltpu.VMEM((CE, AGGW), jnp.float32) for _ in range(2)],
        pltpu.VMEM_SHARED((N_QP, AGGW), jnp.float32),
        [pltpu.SemaphoreType.DMA for _ in range(4)],
        [pltpu.SemaphoreType.DMA for _ in range(4)],
        [pltpu.SemaphoreType.DMA for _ in range(4)],
        [pltpu.SemaphoreType.DMA for _ in range(2)],
    ],
)
def _scatter_agg(ei_hbm, k_hbm, rnd_hbm, outm_hbm, outc_hbm,
                 ib, kv, fv, mv, agg_sh, semd, semk, semf, semsc):
    cid = lax.axis_index("c")
    sid = lax.axis_index("s")
    wid = sid * NC + cid
    base = wid * EPW
    z16 = jnp.zeros((16,), jnp.float32)
    cnt16 = (lax.iota(jnp.int32, 16) == 0).astype(jnp.float32)

    def zrow(i, _):
        for j in range(AGGW // 16):
            mv[0][i, pl.ds(j * 16, 16)] = z16
        return 0

    lax.fori_loop(0, CE, zrow, 0)

    def zcp(i, _):
        pltpu.sync_copy(mv[0], agg_sh.at[pl.ds(sid * QPT + i * CE, CE)])
        return 0

    lax.fori_loop(0, QPT // CE, zcp, 0)

    def crow(i, _):
        mv[0][i, pl.ds(C, 16)] = cnt16
        mv[1][i, pl.ds(C, 16)] = cnt16
        return 0

    lax.fori_loop(0, CE, crow, 0)
    plsc.subcore_barrier()

    def issue_ib(ci, r):
        pltpu.async_copy(ei_hbm.at[:, pl.ds(base + ci * CE, CE)], ib[r],
                         semd[r])

    def issue_k(ci, r):
        pltpu.async_copy(k_hbm.at[pl.ds(base + ci * CE, CE)], kv[r], semk[r])

    def issue_f(r):
        pltpu.async_copy(rnd_hbm.at[ib[r].at[1]], fv[r], semf[r])

    def wait_d(r):
        pltpu.make_async_copy(ei_hbm.at[:, pl.ds(base, CE)], ib[r],
                              semd[r]).wait()

    def wait_k(r):
        pltpu.make_async_copy(k_hbm.at[pl.ds(base, CE)], kv[r], semk[r]).wait()

    def wait_f(r):
        pltpu.make_async_copy(rnd_hbm.at[ib[r].at[1]], fv[r], semf[r]).wait()

    def compute(r, m):
        def erow(h, _):
            for d in range(2):
                e = h * 2 + d
                for g in range(4):
                    prod = (kv[r][e, pl.ds(g * 32, 32)]
                            * fv[r][e, pl.ds(g * 32, 32)])
                    pa, pb = plsc.unpack(prod,
                                         format=plsc.PackFormat.INTERLEAVED)
                    mv[m][e, pl.ds(g * 32, 16)] = pa
                    mv[m][e, pl.ds(g * 32 + 16, 16)] = pb
            return 0

        lax.fori_loop(0, CE // 2, erow, 0)

    def issue_sc(r, m):
        pltpu.async_copy(mv[m], agg_sh.at[ib[r].at[0]], semsc[m], add=True)

    def wait_sc(m):
        pltpu.make_async_copy(mv[m], agg_sh.at[ib[0].at[0]], semsc[m]).wait()

    issue_ib(0, 0)
    issue_ib(1, 1)
    issue_k(0, 0)
    issue_k(1, 1)
    wait_d(0)
    issue_f(0)
    wait_d(1)
    issue_f(1)
    wait_k(0)
    wait_f(0)
    issue_ib(2, 2)
    issue_k(2, 2)
    compute(0, 0)
    issue_sc(0, 0)
    wait_d(2)
    issue_f(2)
    wait_k(1)
    wait_f(1)
    issue_ib(3, 3)
    issue_k(3, 3)
    compute(1, 1)
    issue_sc(1, 1)

    def quad(cj, _):
        for u in range(4):
            ci = 2 + 4 * cj + u
            r = (2 + u) % 4
            nr = (3 + u) % 4
            r2 = u
            m = u % 2

            @pl.when(ci + 1 < NCH)
            def _():
                wait_d(nr)
                issue_f(nr)

            wait_k(r)
            wait_f(r)
            wait_sc(m)

            @pl.when(ci + 2 < NCH)
            def _():
                issue_ib(ci + 2, r2)
                issue_k(ci + 2, r2)

            compute(r, m)
            issue_sc(r, m)
        return 0

    lax.fori_loop(0, (NCH - 2) // 4, quad, 0)
    wait_sc(0)
    wait_sc(1)
    plsc.subcore_barrier()

    def rd(i, _):
        r0 = sid * QPT + i * CE
        pltpu.sync_copy(agg_sh.at[pl.ds(r0, CE), pl.ds(0, C)],
                        outm_hbm.at[cid, pl.ds(r0, CE)])
        pltpu.sync_copy(agg_sh.at[pl.ds(r0, CE), pl.ds(C, 16)],
                        outc_hbm.at[cid, pl.ds(r0, CE)])
        return 0

    lax.fori_loop(0, QPT // CE, rd, 0)


RB = 400


def _proj_body(m0_ref, m1_ref, c0_ref, c1_ref, w1_ref, b1_ref, w2_ref,
               b2_ref, out_ref):
    agg = m0_ref[0] + m1_ref[0]
    cnt = jnp.sum(c0_ref[0] + c1_ref[0], axis=1, keepdims=True)
    decoded = agg / jnp.maximum(cnt, 1.0)
    p = jax.nn.gelu(
        jnp.dot(decoded, w1_ref[...], preferred_element_type=jnp.float32)
        + b1_ref[...][None, :])
    out_ref[...] = (
        jnp.dot(p, w2_ref[...], preferred_element_type=jnp.float32)
        + b2_ref[...][None, :])


def _proj(m, c, w1, b1, w2, b2):
    full = lambda shape: pl.BlockSpec(shape, lambda i: tuple(0 for _ in shape))
    return pl.pallas_call(
        _proj_body,
        grid=(N_Q // RB,),
        in_specs=[
            pl.BlockSpec((1, RB, C), lambda i: (0, i, 0)),
            pl.BlockSpec((1, RB, C), lambda i: (1, i, 0)),
            pl.BlockSpec((1, RB, 16), lambda i: (0, i, 0)),
            pl.BlockSpec((1, RB, 16), lambda i: (1, i, 0)),
            full((C, 256)), full((256,)),
            full((256, C)), full((C,)),
        ],
        out_specs=pl.BlockSpec((RB, C), lambda i: (i, 0)),
        out_shape=jax.ShapeDtypeStruct((N_Q, C), jnp.float32),
    )(m, m, c, c, w1, b1, w2, b2)


def kernel(rndata_flat, phys_pos_query, batch_idx_phys_query,
           latent_tokens_pos, latent_tokens_batch_idx, edge_index,
           Wk1, bk1, Wk2, bk2, Wk3, bk3, Wp1, bp1, Wp2, bp2):
    kin_t = _gather_kin(edge_index, latent_tokens_pos.reshape(-1),
                        phys_pos_query.reshape(-1))
    k = _edge_mlp(kin_t, Wk1.T.astype(jnp.bfloat16), bk1,
                  Wk2.T.astype(jnp.bfloat16), bk2,
                  Wk3.astype(jnp.bfloat16), bk3)
    aggm, aggc = _scatter_agg(edge_index, k,
                              rndata_flat.astype(jnp.bfloat16))
    out = _proj(aggm, aggc, Wp1[_SIGMA], bp1, Wp2, bp2)
    return out
